# Initial kernel scaffold; baseline (speedup 1.0000x reference)
import jax
import jax.numpy as jnp
from jax.experimental import pallas as pl


def kernel(x, edge_index, W1_l, b1_l, W1_r, W2_l, b2_l, W2_r):
    raise NotImplementedError("write your pallas kernel here")


# speedup vs baseline: 4.6239x; 4.6239x over previous
import functools

import jax
import jax.numpy as jnp
from jax import lax
from jax.experimental import pallas as pl
from jax.experimental.pallas import tpu as pltpu
from jax.experimental.pallas import tpu_sc as plsc

N = 10000
E = 320000
D = 128
NW = 32
K = 128
C = 79
PW = C * K
EPAD = NW * PW
NPAD = 10240
ZR = NPAD // 16


def _make_agg():
    mesh = plsc.VectorSubcoreMesh(core_axis_name="c", subcore_axis_name="s")

    @functools.partial(
        pl.kernel,
        out_type=(
            jax.ShapeDtypeStruct((2, NPAD, D), jnp.float32),
            jax.ShapeDtypeStruct((2, NPAD), jnp.float32),
        ),
        mesh=mesh,
        scratch_types=[
            pltpu.VMEM((C, K), jnp.int32),
            pltpu.VMEM((C, K), jnp.int32),
            pltpu.VMEM((K, D), jnp.float32),
            pltpu.VMEM((K,), jnp.float32),
            pltpu.VMEM_SHARED((NPAD, D), jnp.float32),
            pltpu.VMEM_SHARED((NPAD,), jnp.float32),
            pltpu.SemaphoreType.DMA,
        ],
    )
    def agg(feat_hbm, srcs_hbm, dsts_hbm, zrows_hbm, zcnt_hbm,
            agg_out, cnt_out,
            src_v, dst_v, rows_v, ones_v, acc_sh, cnt_sh, sem):
        cid = lax.axis_index("c")
        sid = lax.axis_index("s")
        wid = sid * 2 + cid

        pltpu.sync_copy(zrows_hbm, acc_sh.at[pl.ds(sid * ZR, ZR)])
        pltpu.sync_copy(zcnt_hbm.at[pl.ds(sid * ZR, ZR)],
                        cnt_sh.at[pl.ds(sid * ZR, ZR)])

        pltpu.sync_copy(srcs_hbm.at[wid], src_v)
        pltpu.sync_copy(dsts_hbm.at[wid], dst_v)
        for j in range(K // 16):
            ones_v[pl.ds(j * 16, 16)] = jnp.full((16,), 1.0, jnp.float32)

        plsc.subcore_barrier()

        def body(c, carry):
            pltpu.async_copy(feat_hbm.at[src_v.at[c]], rows_v, sem).wait()
            pltpu.sync_copy(rows_v, acc_sh.at[dst_v.at[c]], add=True)
            pltpu.sync_copy(ones_v, cnt_sh.at[dst_v.at[c]], add=True)
            return carry

        lax.fori_loop(0, C, body, 0)

        plsc.subcore_barrier()

        pltpu.sync_copy(acc_sh.at[pl.ds(sid * ZR, ZR)],
                        agg_out.at[cid, pl.ds(sid * ZR, ZR)])
        pltpu.sync_copy(cnt_sh.at[pl.ds(sid * ZR, ZR)],
                        cnt_out.at[cid, pl.ds(sid * ZR, ZR)])

    return agg


_agg = _make_agg()


def _dense_body(relu, aggp, cntp, x, wl, wr, b, o):
    a = aggp[0, :, :] + aggp[1, :, :]
    cnt = cntp[0, :, :] + cntp[1, :, :]
    mean = a * (1.0 / jnp.maximum(cnt, 1.0))
    acc = jnp.dot(mean, wl[...], preferred_element_type=jnp.float32,
                  precision=lax.Precision.HIGHEST)
    acc += jnp.dot(x[...], wr[...], preferred_element_type=jnp.float32,
                   precision=lax.Precision.HIGHEST)
    acc += b[...]
    if relu:
        acc = jnp.maximum(acc, 0.0)
    o[...] = acc


def _make_dense(relu):
    R = 400
    return pl.pallas_call(
        functools.partial(_dense_body, relu),
        grid=(N // R,),
        in_specs=[
            pl.BlockSpec((2, R, D), lambda i: (0, i, 0)),
            pl.BlockSpec((2, R, 1), lambda i: (0, i, 0)),
            pl.BlockSpec((R, D), lambda i: (i, 0)),
            pl.BlockSpec((D, D), lambda i: (0, 0)),
            pl.BlockSpec((D, D), lambda i: (0, 0)),
            pl.BlockSpec((1, D), lambda i: (0, 0)),
        ],
        out_specs=pl.BlockSpec((R, D), lambda i: (i, 0)),
        out_shape=jax.ShapeDtypeStruct((N, D), jnp.float32),
    )


_dense_relu = _make_dense(True)
_dense_lin = _make_dense(False)


def kernel(x, edge_index, W1_l, b1_l, W1_r, W2_l, b2_l, W2_r):
    src = edge_index[0].astype(jnp.int32)
    dst = edge_index[1].astype(jnp.int32)
    pad = EPAD - E
    src_p = jnp.concatenate([src, jnp.zeros((pad,), jnp.int32)]).reshape(NW, C, K)
    dst_p = jnp.concatenate([dst, jnp.full((pad,), N, jnp.int32)]).reshape(NW, C, K)
    zrows = jnp.zeros((ZR, D), jnp.float32)
    zcnt = jnp.zeros((NPAD,), jnp.float32)

    agg1, cnt1 = _agg(x, src_p, dst_p, zrows, zcnt)
    h = _dense_relu(agg1, cnt1[:, :, None], x, W1_l, W1_r, b1_l.reshape(1, D))
    agg2, _ = _agg(h, src_p, dst_p, zrows, zcnt)
    out = _dense_lin(agg2, cnt1[:, :, None], h, W2_l, W2_r, b2_l.reshape(1, D))
    return out
